# baseline (speedup 1.0000x reference)
import functools

import jax
import jax.numpy as jnp
from jax.experimental import pallas as pl
from jax.experimental.pallas import tpu as pltpu

_N = 50000
_A = 512
_BLK = 5000


def _residual_linear_kernel(x_ref, w_ref, b_ref, o_ref, xb_ref, wb_ref):
    xb = x_ref[...]
    xb_ref[...] = xb.astype(jnp.bfloat16)
    wb_ref[...] = w_ref[...].astype(jnp.bfloat16)
    acc = jax.lax.dot_general(
        xb_ref[...], wb_ref[...],
        dimension_numbers=(((1,), (1,)), ((), ())),
        preferred_element_type=jnp.float32,
    )
    o_ref[...] = xb + 0.1 * acc + 0.1 * b_ref[...]


@functools.partial(jax.jit, static_argnames=())
def kernel(x, W, b):
    b2 = b.reshape(1, _A)
    grid = (pl.cdiv(_N, _BLK),)
    return pl.pallas_call(
        _residual_linear_kernel,
        grid=grid,
        in_specs=[
            pl.BlockSpec((_BLK, _A), lambda i: (i, 0)),
            pl.BlockSpec((_A, _A), lambda i: (0, 0)),
            pl.BlockSpec((1, _A), lambda i: (0, 0)),
        ],
        out_specs=pl.BlockSpec((_BLK, _A), lambda i: (i, 0)),
        out_shape=jax.ShapeDtypeStruct((_N, _A), jnp.float32),
        scratch_shapes=[
            pltpu.VMEM((_BLK, _A), jnp.bfloat16),
            pltpu.VMEM((_A, _A), jnp.bfloat16),
        ],
    )(x, W, b2)


# speedup vs baseline: 1.0390x; 1.0390x over previous
import functools

import jax
import jax.numpy as jnp
from jax.experimental import pallas as pl
from jax.experimental.pallas import tpu as pltpu

_N = 50000
_A = 512
_CHUNK = 2000
_NBUF = 5
_NSTEPS = _N // _CHUNK


def _pipelined_kernel(x_hbm, w_ref, b_ref, o_hbm, xbuf, obuf, in_sems, out_sems):
    w = w_ref[...]
    bias = b_ref[...]

    def in_copy(i, s):
        return pltpu.make_async_copy(
            x_hbm.at[pl.ds(i * _CHUNK, _CHUNK), :], xbuf.at[s], in_sems.at[s])

    def out_copy(i, s):
        return pltpu.make_async_copy(
            obuf.at[s], o_hbm.at[pl.ds(i * _CHUNK, _CHUNK), :], out_sems.at[s])

    for s in range(_NBUF):
        in_copy(s, s).start()

    for i in range(_NSTEPS):
        s = i % _NBUF
        in_copy(i, s).wait()
        if i >= _NBUF:
            out_copy(i - _NBUF, s).wait()
        xb = xbuf[s]
        acc = jax.lax.dot_general(
            xb, w,
            dimension_numbers=(((1,), (1,)), ((), ())),
            preferred_element_type=jnp.float32,
        )
        obuf[s] = xb + 0.1 * acc + 0.1 * bias
        out_copy(i, s).start()
        if i + _NBUF < _NSTEPS:
            in_copy(i + _NBUF, s).start()

    for i in range(_NSTEPS - _NBUF, _NSTEPS):
        out_copy(i, i % _NBUF).wait()


@functools.partial(jax.jit, static_argnames=())
def kernel(x, W, b):
    b2 = b.reshape(1, _A)
    return pl.pallas_call(
        _pipelined_kernel,
        in_specs=[
            pl.BlockSpec(memory_space=pltpu.HBM),
            pl.BlockSpec((_A, _A), lambda: (0, 0)),
            pl.BlockSpec((1, _A), lambda: (0, 0)),
        ],
        out_specs=pl.BlockSpec(memory_space=pltpu.HBM),
        out_shape=jax.ShapeDtypeStruct((_N, _A), jnp.float32),
        scratch_shapes=[
            pltpu.VMEM((_NBUF, _CHUNK, _A), jnp.float32),
            pltpu.VMEM((_NBUF, _CHUNK, _A), jnp.float32),
            pltpu.SemaphoreType.DMA((_NBUF,)),
            pltpu.SemaphoreType.DMA((_NBUF,)),
        ],
    )(x, W, b2)
